# Initial kernel scaffold; baseline (speedup 1.0000x reference)
import jax
import jax.numpy as jnp
from jax.experimental import pallas as pl


def kernel(x, lower_neighborhood, upper_neighborhood, W_irr, W_sol):
    raise NotImplementedError("write your pallas kernel here")


# speedup vs baseline: 1.0325x; 1.0325x over previous
import jax
import jax.numpy as jnp
from jax.experimental import pallas as pl
from jax.experimental.pallas import tpu as pltpu

_BM = 200


def _body(x_ref, l_ref, u_ref, wi_ref, ws_ref, out_ref):
    lb = l_ref[...].astype(jnp.bfloat16)
    ub = u_ref[...].astype(jnp.bfloat16)
    t_l = jnp.dot(lb, x_ref[...], preferred_element_type=jnp.float32)
    t_u = jnp.dot(ub, x_ref[...], preferred_element_type=jnp.float32)
    t = (jnp.dot(t_l, wi_ref[...], preferred_element_type=jnp.float32)
         + jnp.dot(t_u, ws_ref[...], preferred_element_type=jnp.float32))
    out_ref[...] = jnp.maximum(t, 0.0)


def _run(x, lower, upper, w_irr, w_sol, bm):
    n, d = x.shape
    d_out = w_irr.shape[1]
    xb = x.astype(jnp.bfloat16)
    return pl.pallas_call(
        _body,
        grid=(n // bm,),
        in_specs=[
            pl.BlockSpec((n, d), lambda m: (0, 0)),
            pl.BlockSpec((bm, n), lambda m: (m, 0)),
            pl.BlockSpec((bm, n), lambda m: (m, 0)),
            pl.BlockSpec((d, d_out), lambda m: (0, 0)),
            pl.BlockSpec((d, d_out), lambda m: (0, 0)),
        ],
        out_specs=pl.BlockSpec((bm, d_out), lambda m: (m, 0)),
        out_shape=jax.ShapeDtypeStruct((n, d_out), jnp.float32),
        compiler_params=pltpu.CompilerParams(
            dimension_semantics=("parallel",),
        ),
    )(xb, lower, upper, w_irr, w_sol)


def kernel(x, lower_neighborhood, upper_neighborhood, W_irr, W_sol):
    return _run(x, lower_neighborhood, upper_neighborhood, W_irr, W_sol, _BM)


# speedup vs baseline: 1.0436x; 1.0108x over previous
import jax
import jax.numpy as jnp
from jax.experimental import pallas as pl
from jax.experimental.pallas import tpu as pltpu

_BM = 200


def _body(x_ref, l_ref, u_ref, wi_ref, ws_ref, out_ref):
    xb = x_ref[...].astype(jnp.bfloat16)
    lb = l_ref[...].astype(jnp.bfloat16)
    ub = u_ref[...].astype(jnp.bfloat16)
    t_l = jnp.dot(lb, xb, preferred_element_type=jnp.float32)
    t_u = jnp.dot(ub, xb, preferred_element_type=jnp.float32)
    t = (jnp.dot(t_l, wi_ref[...], preferred_element_type=jnp.float32)
         + jnp.dot(t_u, ws_ref[...], preferred_element_type=jnp.float32))
    out_ref[...] = jnp.maximum(t, 0.0)


def _run(x, lower, upper, w_irr, w_sol, bm):
    n, d = x.shape
    d_out = w_irr.shape[1]
    return pl.pallas_call(
        _body,
        grid=(n // bm,),
        in_specs=[
            pl.BlockSpec((n, d), lambda m: (0, 0)),
            pl.BlockSpec((bm, n), lambda m: (m, 0)),
            pl.BlockSpec((bm, n), lambda m: (m, 0)),
            pl.BlockSpec((d, d_out), lambda m: (0, 0)),
            pl.BlockSpec((d, d_out), lambda m: (0, 0)),
        ],
        out_specs=pl.BlockSpec((bm, d_out), lambda m: (m, 0)),
        out_shape=jax.ShapeDtypeStruct((n, d_out), jnp.float32),
        compiler_params=pltpu.CompilerParams(
            dimension_semantics=("parallel",),
        ),
    )(x, lower, upper, w_irr, w_sol)


def kernel(x, lower_neighborhood, upper_neighborhood, W_irr, W_sol):
    return _run(x, lower_neighborhood, upper_neighborhood, W_irr, W_sol, _BM)
